# Initial kernel scaffold; baseline (speedup 1.0000x reference)
import jax
import jax.numpy as jnp
from jax.experimental import pallas as pl


def kernel(x, edge_index, edge_type, W1, root1, b1, W2, root2, b2, fc_w, fc_b):
    raise NotImplementedError("write your pallas kernel here")


# speedup vs baseline: 5.5725x; 5.5725x over previous
import functools

import jax
import jax.numpy as jnp
from jax import lax
from jax.experimental import pallas as pl
from jax.experimental.pallas import tpu as pltpu
from jax.experimental.pallas import tpu_sc as plsc

_N = 10000
_E = 320000
_F = 128
_H = 128
_R = 3
_NC = 2
_NS = 16
_HALF = _N // 2
_ROWS = _R * _HALF
_APAD = 15360
_TRASH = 15350
_K = 80
_EPT = _E // (_NC * _NS)
_EPS = _E // _NS

_sc_mesh = plsc.VectorSubcoreMesh(
    core_axis_name="c", subcore_axis_name="s", num_cores=_NC, num_subcores=_NS
)


@functools.partial(
    pl.kernel,
    out_type=(
        jax.ShapeDtypeStruct((_E,), jnp.int32),
        jax.ShapeDtypeStruct((_NC * _E,), jnp.int32),
        jax.ShapeDtypeStruct((_NC * _APAD,), jnp.float32),
    ),
    mesh=_sc_mesh,
    scratch_types=[
        pltpu.VMEM((_K,), jnp.int32),
        pltpu.VMEM((_K,), jnp.int32),
        pltpu.VMEM((_K,), jnp.int32),
        pltpu.VMEM((_K,), jnp.int32),
        pltpu.VMEM((_K,), jnp.int32),
        pltpu.VMEM((_K,), jnp.int32),
        pltpu.VMEM((_K,), jnp.int32),
        pltpu.VMEM((_K,), jnp.float32),
        pltpu.VMEM((960,), jnp.float32),
        pltpu.VMEM_SHARED((_APAD,), jnp.float32),
    ],
)
def _prep(src_hbm, dst_hbm, et_hbm, gidx_hbm, sidx_hbm, cnt_hbm,
          srcv, dstv, etv, giv, s0v, s1v, segv, onesv, zv, cnt_acc):
    c = lax.axis_index("c")
    s = lax.axis_index("s")
    wid = s * _NC + c

    @pl.loop(0, 60)
    def _(i):
        zv[pl.ds(i * 16, 16)] = jnp.zeros((16,), jnp.float32)

    @pl.loop(0, _K // 16)
    def _(i):
        onesv[pl.ds(i * 16, 16)] = jnp.ones((16,), jnp.float32)

    pltpu.sync_copy(zv, cnt_acc.at[pl.ds(s * 960, 960)])
    plsc.subcore_barrier()

    @pl.loop(0, _EPT // _K)
    def _(ck):
        off = wid * _EPT + ck * _K
        pltpu.sync_copy(src_hbm.at[pl.ds(off, _K)], srcv)
        pltpu.sync_copy(dst_hbm.at[pl.ds(off, _K)], dstv)
        pltpu.sync_copy(et_hbm.at[pl.ds(off, _K)], etv)
        for g in range(_K // 16):
            sl = pl.ds(g * 16, 16)
            sv = srcv[sl]
            dv = dstv[sl]
            tv = etv[sl]
            giv[sl] = tv * _N + sv
            in0 = dv < _HALF
            s0v[sl] = jnp.where(in0, tv * _HALF + dv, _TRASH)
            s1v[sl] = jnp.where(in0, _TRASH, tv * _HALF + (dv - _HALF))
        pltpu.sync_copy(giv, gidx_hbm.at[pl.ds(off, _K)])
        pltpu.sync_copy(s0v, sidx_hbm.at[pl.ds(off, _K)])
        pltpu.sync_copy(s1v, sidx_hbm.at[pl.ds(_E + off, _K)])

    base_lo = c * _HALF

    @pl.loop(0, _EPS // _K)
    def _(ck):
        off = s * _EPS + ck * _K
        pltpu.sync_copy(dst_hbm.at[pl.ds(off, _K)], dstv)
        pltpu.sync_copy(et_hbm.at[pl.ds(off, _K)], etv)
        for g in range(_K // 16):
            sl = pl.ds(g * 16, 16)
            dv = dstv[sl] - base_lo
            tv = etv[sl]
            own = (dv >= 0) & (dv < _HALF)
            segv[sl] = jnp.where(own, dv * _R + tv, _TRASH)
        pltpu.sync_copy(onesv, cnt_acc.at[segv], add=True)

    plsc.subcore_barrier()
    pltpu.sync_copy(cnt_acc.at[pl.ds(s * 960, 960)], zv)
    pltpu.sync_copy(zv, cnt_hbm.at[pl.ds(c * _APAD + s * 960, 960)])


_HS = _H // 2


@functools.partial(
    pl.kernel,
    out_type=jax.ShapeDtypeStruct((_NC, _APAD, _HS), jnp.float32),
    mesh=_sc_mesh,
    scratch_types=[
        pltpu.VMEM((_K,), jnp.int32),
        pltpu.VMEM((_K,), jnp.int32),
        pltpu.VMEM((_K, _HS), jnp.float32),
        pltpu.VMEM((160, _HS), jnp.float32),
        pltpu.VMEM_SHARED((_APAD, _HS), jnp.float32),
        pltpu.SemaphoreType.DMA,
    ],
    compiler_params=pltpu.CompilerParams(use_tc_tiling_on_sc=False),
)
def _agg(xw_hbm, gidx_hbm, sidx_hbm, out_hbm, giv, siv, rows, zrow, acc, sem):
    c = lax.axis_index("c")
    s = lax.axis_index("s")

    @pl.loop(0, 160)
    def _(r):
        for j in range(_HS // 16):
            zrow[r, pl.ds(j * 16, 16)] = jnp.zeros((16,), jnp.float32)

    @pl.loop(0, 6)
    def _(i):
        pltpu.sync_copy(zrow, acc.at[pl.ds(s * 960 + i * 160, 160)])

    plsc.subcore_barrier()

    @pl.loop(0, _EPS // _K)
    def _(ck):
        off = s * _EPS + ck * _K
        pltpu.sync_copy(gidx_hbm.at[pl.ds(off, _K)], giv)
        pltpu.sync_copy(sidx_hbm.at[pl.ds(c * _E + off, _K)], siv)
        pltpu.async_copy(xw_hbm.at[giv], rows, sem).wait()
        pltpu.sync_copy(rows, acc.at[siv], add=True)

    plsc.subcore_barrier()
    @pl.loop(0, 6)
    def _(i):
        pltpu.sync_copy(acc.at[pl.ds(s * 960 + i * 160, 160)], zrow)
        pltpu.sync_copy(zrow, out_hbm.at[c, pl.ds(s * 960 + i * 160, 160)])


_BN = 2000


def _xw_body(x_ref, w_ref, o1_ref, o2_ref):
    res = jnp.dot(x_ref[...], w_ref[0], preferred_element_type=jnp.float32)
    o1_ref[0] = res[:, :_HS]
    o2_ref[0] = res[:, _HS:]


def _xw(x, W):
    return pl.pallas_call(
        _xw_body,
        grid=(_R, _N // _BN),
        in_specs=[
            pl.BlockSpec((_BN, _F), lambda r, i: (i, 0)),
            pl.BlockSpec((1, _F, _H), lambda r, i: (r, 0, 0)),
        ],
        out_specs=[
            pl.BlockSpec((1, _BN, _HS), lambda r, i: (r, i, 0)),
            pl.BlockSpec((1, _BN, _HS), lambda r, i: (r, i, 0)),
        ],
        out_shape=(
            jax.ShapeDtypeStruct((_R, _N, _HS), jnp.float32),
            jax.ShapeDtypeStruct((_R, _N, _HS), jnp.float32),
        ),
    )(x, W)


_BJ = 1000
_NBJ = _HALF // _BJ


def _norm_root(al0, al1, al2, ah0, ah1, ah2, cnt_ref, x_ref, root_ref, b_ref):
    inv = 1.0 / jnp.maximum(cnt_ref[...], 1.0)
    agg_lo = (al0[0] * inv[:, 0:1] + al1[0] * inv[:, 1:2]
              + al2[0] * inv[:, 2:3])
    agg_hi = (ah0[0] * inv[:, 0:1] + ah1[0] * inv[:, 1:2]
              + ah2[0] * inv[:, 2:3])
    agg = jnp.concatenate([agg_lo, agg_hi], axis=1)
    h = agg + jnp.dot(x_ref[...], root_ref[...],
                      preferred_element_type=jnp.float32) + b_ref[...]
    return jnp.maximum(h, 0.0)


def _k2a_body(al0, al1, al2, ah0, ah1, ah2, cnt_ref, x_ref, root_ref, b_ref,
              o_ref):
    o_ref[...] = _norm_root(al0, al1, al2, ah0, ah1, ah2,
                            cnt_ref, x_ref, root_ref, b_ref)


def _k2b_body(al0, al1, al2, ah0, ah1, ah2, cnt_ref, x_ref, root_ref, b_ref,
              fcw_ref, fcb_ref, o_ref):
    h = _norm_root(al0, al1, al2, ah0, ah1, ah2,
                   cnt_ref, x_ref, root_ref, b_ref)
    z = jnp.dot(h, fcw_ref[...], preferred_element_type=jnp.float32)
    o_ref[...] = jax.nn.sigmoid(z + fcb_ref[...])


def _acc_specs():
    return [
        pl.BlockSpec((1, _BJ, _HS), lambda c, j, r=r: (c, r * _NBJ + j, 0))
        for r in range(_R)
    ]


def _k2a(acc_lo, acc_hi, cnt3, x, root, b):
    return pl.pallas_call(
        _k2a_body,
        grid=(_NC, _NBJ),
        in_specs=_acc_specs() * 2 + [
            pl.BlockSpec((_BJ, _R), lambda c, j: (c * _NBJ + j, 0)),
            pl.BlockSpec((_BJ, _H), lambda c, j: (c * _NBJ + j, 0)),
            pl.BlockSpec((_H, _H), lambda c, j: (0, 0)),
            pl.BlockSpec((1, _H), lambda c, j: (0, 0)),
        ],
        out_specs=pl.BlockSpec((_BJ, _H), lambda c, j: (c * _NBJ + j, 0)),
        out_shape=jax.ShapeDtypeStruct((_N, _H), jnp.float32),
    )(acc_lo, acc_lo, acc_lo, acc_hi, acc_hi, acc_hi, cnt3, x, root, b)


def _k2b(acc_lo, acc_hi, cnt3, x, root, b, fc_w, fc_b):
    return pl.pallas_call(
        _k2b_body,
        grid=(_NC, _NBJ),
        in_specs=_acc_specs() * 2 + [
            pl.BlockSpec((_BJ, _R), lambda c, j: (c * _NBJ + j, 0)),
            pl.BlockSpec((_BJ, _H), lambda c, j: (c * _NBJ + j, 0)),
            pl.BlockSpec((_H, _H), lambda c, j: (0, 0)),
            pl.BlockSpec((1, _H), lambda c, j: (0, 0)),
            pl.BlockSpec((_H, 1), lambda c, j: (0, 0)),
            pl.BlockSpec((1, 1), lambda c, j: (0, 0)),
        ],
        out_specs=pl.BlockSpec((_BJ, 1), lambda c, j: (c * _NBJ + j, 0)),
        out_shape=jax.ShapeDtypeStruct((_N, 1), jnp.float32),
    )(acc_lo, acc_lo, acc_lo, acc_hi, acc_hi, acc_hi, cnt3, x, root, b,
      fc_w, fc_b)


def kernel(x, edge_index, edge_type, W1, root1, b1, W2, root2, b2, fc_w, fc_b):
    gidx, sidx, cnt = _prep(edge_index[0], edge_index[1], edge_type)
    cnt3 = cnt.reshape(_NC, _APAD)[:, :_ROWS].reshape(_N, _R)

    xw1_lo, xw1_hi = _xw(x, W1)
    acc1_lo = _agg(xw1_lo.reshape(_R * _N, _HS), gidx, sidx)
    acc1_hi = _agg(xw1_hi.reshape(_R * _N, _HS), gidx, sidx)
    h1 = _k2a(acc1_lo, acc1_hi, cnt3, x, root1, b1.reshape(1, _H))

    xw2_lo, xw2_hi = _xw(h1, W2)
    acc2_lo = _agg(xw2_lo.reshape(_R * _N, _HS), gidx, sidx)
    acc2_hi = _agg(xw2_hi.reshape(_R * _N, _HS), gidx, sidx)
    return _k2b(acc2_lo, acc2_hi, cnt3, h1, root2, b2.reshape(1, _H),
                fc_w, fc_b.reshape(1, 1))


# speedup vs baseline: 8.0374x; 1.4423x over previous
import functools

import jax
import jax.numpy as jnp
from jax import lax
from jax.experimental import pallas as pl
from jax.experimental.pallas import tpu as pltpu
from jax.experimental.pallas import tpu_sc as plsc

_N = 10000
_E = 320000
_F = 128
_H = 128
_R = 3
_NC = 2
_NS = 16
_HALF = _N // 2
_ROWS = _R * _HALF
_APAD = 15360
_TRASH = 15350
_K = 80
_EPT = _E // (_NC * _NS)
_EPS = _E // _NS

_sc_mesh = plsc.VectorSubcoreMesh(
    core_axis_name="c", subcore_axis_name="s", num_cores=_NC, num_subcores=_NS
)


@functools.partial(
    pl.kernel,
    out_type=(
        jax.ShapeDtypeStruct((_E,), jnp.int32),
        jax.ShapeDtypeStruct((_NC * _E,), jnp.int32),
        jax.ShapeDtypeStruct((_NC * _APAD,), jnp.float32),
    ),
    mesh=_sc_mesh,
    scratch_types=[
        pltpu.VMEM((_K,), jnp.int32),
        pltpu.VMEM((_K,), jnp.int32),
        pltpu.VMEM((_K,), jnp.int32),
        pltpu.VMEM((_K,), jnp.int32),
        pltpu.VMEM((_K,), jnp.int32),
        pltpu.VMEM((_K,), jnp.int32),
        pltpu.VMEM((_K,), jnp.int32),
        pltpu.VMEM((_K,), jnp.float32),
        pltpu.VMEM((960,), jnp.float32),
        pltpu.VMEM_SHARED((_APAD,), jnp.float32),
    ],
)
def _prep(src_hbm, dst_hbm, et_hbm, gidx_hbm, sidx_hbm, cnt_hbm,
          srcv, dstv, etv, giv, s0v, s1v, segv, onesv, zv, cnt_acc):
    c = lax.axis_index("c")
    s = lax.axis_index("s")
    wid = s * _NC + c

    @pl.loop(0, 60)
    def _(i):
        zv[pl.ds(i * 16, 16)] = jnp.zeros((16,), jnp.float32)

    @pl.loop(0, _K // 16)
    def _(i):
        onesv[pl.ds(i * 16, 16)] = jnp.ones((16,), jnp.float32)

    pltpu.sync_copy(zv, cnt_acc.at[pl.ds(s * 960, 960)])
    plsc.subcore_barrier()

    @pl.loop(0, _EPT // _K)
    def _(ck):
        off = wid * _EPT + ck * _K
        pltpu.sync_copy(src_hbm.at[pl.ds(off, _K)], srcv)
        pltpu.sync_copy(dst_hbm.at[pl.ds(off, _K)], dstv)
        pltpu.sync_copy(et_hbm.at[pl.ds(off, _K)], etv)
        for g in range(_K // 16):
            sl = pl.ds(g * 16, 16)
            sv = srcv[sl]
            dv = dstv[sl]
            tv = etv[sl]
            giv[sl] = tv * _N + sv
            in0 = dv < _HALF
            s0v[sl] = jnp.where(in0, tv * _HALF + dv, _TRASH)
            s1v[sl] = jnp.where(in0, _TRASH, tv * _HALF + (dv - _HALF))
        pltpu.sync_copy(giv, gidx_hbm.at[pl.ds(off, _K)])
        pltpu.sync_copy(s0v, sidx_hbm.at[pl.ds(off, _K)])
        pltpu.sync_copy(s1v, sidx_hbm.at[pl.ds(_E + off, _K)])

    base_lo = c * _HALF

    @pl.loop(0, _EPS // _K)
    def _(ck):
        off = s * _EPS + ck * _K
        pltpu.sync_copy(dst_hbm.at[pl.ds(off, _K)], dstv)
        pltpu.sync_copy(et_hbm.at[pl.ds(off, _K)], etv)
        for g in range(_K // 16):
            sl = pl.ds(g * 16, 16)
            dv = dstv[sl] - base_lo
            tv = etv[sl]
            own = (dv >= 0) & (dv < _HALF)
            segv[sl] = jnp.where(own, dv * _R + tv, _TRASH)
        pltpu.sync_copy(onesv, cnt_acc.at[segv], add=True)

    plsc.subcore_barrier()
    pltpu.sync_copy(cnt_acc.at[pl.ds(s * 960, 960)], zv)
    pltpu.sync_copy(zv, cnt_hbm.at[pl.ds(c * _APAD + s * 960, 960)])


_HS = _H // 2


@functools.partial(
    pl.kernel,
    out_type=jax.ShapeDtypeStruct((_NC, _APAD, _HS), jnp.float32),
    mesh=_sc_mesh,
    scratch_types=[
        pltpu.VMEM((_K,), jnp.int32),
        pltpu.VMEM((_K,), jnp.int32),
        pltpu.VMEM((_K, _HS), jnp.float32),
        pltpu.VMEM((_K,), jnp.int32),
        pltpu.VMEM((_K,), jnp.int32),
        pltpu.VMEM((_K, _HS), jnp.float32),
        pltpu.VMEM((160, _HS), jnp.float32),
        pltpu.VMEM_SHARED((_APAD, _HS), jnp.float32),
        pltpu.SemaphoreType.DMA,
        pltpu.SemaphoreType.DMA,
    ],
    compiler_params=pltpu.CompilerParams(use_tc_tiling_on_sc=False),
)
def _agg(xw_hbm, gidx_hbm, sidx_hbm, out_hbm,
         giv0, siv0, rows0, giv1, siv1, rows1, zrow, acc, sem0, sem1):
    c = lax.axis_index("c")
    s = lax.axis_index("s")

    @pl.loop(0, 160)
    def _(r):
        for j in range(_HS // 16):
            zrow[r, pl.ds(j * 16, 16)] = jnp.zeros((16,), jnp.float32)

    @pl.loop(0, 6)
    def _(i):
        pltpu.sync_copy(zrow, acc.at[pl.ds(s * 960 + i * 160, 160)])

    plsc.subcore_barrier()

    base = s * _EPS
    nch = _EPS // _K

    def _load_and_fire(off, giv, siv, sem, rows):
        pltpu.sync_copy(gidx_hbm.at[pl.ds(off, _K)], giv)
        pltpu.sync_copy(sidx_hbm.at[pl.ds(c * _E + off, _K)], siv)
        pltpu.async_copy(xw_hbm.at[giv], rows, sem)

    def _drain_and_scatter(giv, siv, sem, rows):
        pltpu.make_async_copy(xw_hbm.at[giv], rows, sem).wait()
        pltpu.sync_copy(rows, acc.at[siv], add=True)

    _load_and_fire(base, giv0, siv0, sem0, rows0)

    @pl.loop(0, nch - 1)
    def _(i):
        off_next = base + (i + 1) * _K

        @pl.when(i % 2 == 0)
        def _():
            _load_and_fire(off_next, giv1, siv1, sem1, rows1)
            _drain_and_scatter(giv0, siv0, sem0, rows0)

        @pl.when(i % 2 == 1)
        def _():
            _load_and_fire(off_next, giv0, siv0, sem0, rows0)
            _drain_and_scatter(giv1, siv1, sem1, rows1)

    _drain_and_scatter(giv1, siv1, sem1, rows1)

    plsc.subcore_barrier()
    @pl.loop(0, 6)
    def _(i):
        pltpu.sync_copy(acc.at[pl.ds(s * 960 + i * 160, 160)], zrow)
        pltpu.sync_copy(zrow, out_hbm.at[c, pl.ds(s * 960 + i * 160, 160)])


_BN = 2000


def _xw_body(x_ref, w_ref, o1_ref, o2_ref):
    res = jnp.dot(x_ref[...], w_ref[0], preferred_element_type=jnp.float32)
    o1_ref[0] = res[:, :_HS]
    o2_ref[0] = res[:, _HS:]


def _xw(x, W):
    return pl.pallas_call(
        _xw_body,
        grid=(_R, _N // _BN),
        in_specs=[
            pl.BlockSpec((_BN, _F), lambda r, i: (i, 0)),
            pl.BlockSpec((1, _F, _H), lambda r, i: (r, 0, 0)),
        ],
        out_specs=[
            pl.BlockSpec((1, _BN, _HS), lambda r, i: (r, i, 0)),
            pl.BlockSpec((1, _BN, _HS), lambda r, i: (r, i, 0)),
        ],
        out_shape=(
            jax.ShapeDtypeStruct((_R, _N, _HS), jnp.float32),
            jax.ShapeDtypeStruct((_R, _N, _HS), jnp.float32),
        ),
    )(x, W)


_BJ = 1000
_NBJ = _HALF // _BJ


def _norm_root(al0, al1, al2, ah0, ah1, ah2, cnt_ref, x_ref, root_ref, b_ref):
    inv = 1.0 / jnp.maximum(cnt_ref[...], 1.0)
    agg_lo = (al0[0] * inv[:, 0:1] + al1[0] * inv[:, 1:2]
              + al2[0] * inv[:, 2:3])
    agg_hi = (ah0[0] * inv[:, 0:1] + ah1[0] * inv[:, 1:2]
              + ah2[0] * inv[:, 2:3])
    agg = jnp.concatenate([agg_lo, agg_hi], axis=1)
    h = agg + jnp.dot(x_ref[...], root_ref[...],
                      preferred_element_type=jnp.float32) + b_ref[...]
    return jnp.maximum(h, 0.0)


def _k2a_body(al0, al1, al2, ah0, ah1, ah2, cnt_ref, x_ref, root_ref, b_ref,
              o_ref):
    o_ref[...] = _norm_root(al0, al1, al2, ah0, ah1, ah2,
                            cnt_ref, x_ref, root_ref, b_ref)


def _k2b_body(al0, al1, al2, ah0, ah1, ah2, cnt_ref, x_ref, root_ref, b_ref,
              fcw_ref, fcb_ref, o_ref):
    h = _norm_root(al0, al1, al2, ah0, ah1, ah2,
                   cnt_ref, x_ref, root_ref, b_ref)
    z = jnp.dot(h, fcw_ref[...], preferred_element_type=jnp.float32)
    o_ref[...] = jax.nn.sigmoid(z + fcb_ref[...])


def _acc_specs():
    return [
        pl.BlockSpec((1, _BJ, _HS), lambda c, j, r=r: (c, r * _NBJ + j, 0))
        for r in range(_R)
    ]


def _k2a(acc_lo, acc_hi, cnt3, x, root, b):
    return pl.pallas_call(
        _k2a_body,
        grid=(_NC, _NBJ),
        in_specs=_acc_specs() * 2 + [
            pl.BlockSpec((_BJ, _R), lambda c, j: (c * _NBJ + j, 0)),
            pl.BlockSpec((_BJ, _H), lambda c, j: (c * _NBJ + j, 0)),
            pl.BlockSpec((_H, _H), lambda c, j: (0, 0)),
            pl.BlockSpec((1, _H), lambda c, j: (0, 0)),
        ],
        out_specs=pl.BlockSpec((_BJ, _H), lambda c, j: (c * _NBJ + j, 0)),
        out_shape=jax.ShapeDtypeStruct((_N, _H), jnp.float32),
    )(acc_lo, acc_lo, acc_lo, acc_hi, acc_hi, acc_hi, cnt3, x, root, b)


def _k2b(acc_lo, acc_hi, cnt3, x, root, b, fc_w, fc_b):
    return pl.pallas_call(
        _k2b_body,
        grid=(_NC, _NBJ),
        in_specs=_acc_specs() * 2 + [
            pl.BlockSpec((_BJ, _R), lambda c, j: (c * _NBJ + j, 0)),
            pl.BlockSpec((_BJ, _H), lambda c, j: (c * _NBJ + j, 0)),
            pl.BlockSpec((_H, _H), lambda c, j: (0, 0)),
            pl.BlockSpec((1, _H), lambda c, j: (0, 0)),
            pl.BlockSpec((_H, 1), lambda c, j: (0, 0)),
            pl.BlockSpec((1, 1), lambda c, j: (0, 0)),
        ],
        out_specs=pl.BlockSpec((_BJ, 1), lambda c, j: (c * _NBJ + j, 0)),
        out_shape=jax.ShapeDtypeStruct((_N, 1), jnp.float32),
    )(acc_lo, acc_lo, acc_lo, acc_hi, acc_hi, acc_hi, cnt3, x, root, b,
      fc_w, fc_b)


def kernel(x, edge_index, edge_type, W1, root1, b1, W2, root2, b2, fc_w, fc_b):
    gidx, sidx, cnt = _prep(edge_index[0], edge_index[1], edge_type)
    cnt3 = cnt.reshape(_NC, _APAD)[:, :_ROWS].reshape(_N, _R)

    xw1_lo, xw1_hi = _xw(x, W1)
    acc1_lo = _agg(xw1_lo.reshape(_R * _N, _HS), gidx, sidx)
    acc1_hi = _agg(xw1_hi.reshape(_R * _N, _HS), gidx, sidx)
    h1 = _k2a(acc1_lo, acc1_hi, cnt3, x, root1, b1.reshape(1, _H))

    xw2_lo, xw2_hi = _xw(h1, W2)
    acc2_lo = _agg(xw2_lo.reshape(_R * _N, _HS), gidx, sidx)
    acc2_hi = _agg(xw2_hi.reshape(_R * _N, _HS), gidx, sidx)
    return _k2b(acc2_lo, acc2_hi, cnt3, h1, root2, b2.reshape(1, _H),
                fc_w, fc_b.reshape(1, 1))
